# baseline (speedup 1.0000x reference)
import functools

import jax
import jax.numpy as jnp
from jax.experimental import pallas as pl

P = 512
H = 64
E = 32
N = P * P


def _lstm_slab_kernel(corr_ref, nei_ref, ht_ref, ct_ref,
                      wemb_ref, bemb_ref, wih_ref, whh_ref, bias_ref,
                      ho_ref, co_ref):
    g = ht_ref.shape[0]
    for j in range(g):
        corr = corr_ref[j]
        ht = ht_ref[j]
        ct = ct_ref[j]
        emb = jnp.maximum(
            jnp.dot(wemb_ref[...], corr, preferred_element_type=jnp.float32)
            + bemb_ref[...], 0.0)
        gates = (jnp.dot(wih_ref[...], emb, preferred_element_type=jnp.float32)
                 + jnp.dot(whh_ref[...], ht, preferred_element_type=jnp.float32)
                 + bias_ref[...])
        i = 0.5 * jnp.tanh(gates[0 * H:1 * H, :]) + 0.5
        f = 0.5 * jnp.tanh(gates[1 * H:2 * H, :]) + 0.5
        gg = jnp.tanh(gates[2 * H:3 * H, :])
        o = 0.5 * jnp.tanh(gates[3 * H:4 * H, :]) + 0.5
        c_new = f * ct + i * gg
        h_new = o * jnp.tanh(c_new)
        m = nei_ref[j] > 0
        ho_ref[j] = jnp.where(m, h_new, ht)
        co_ref[j] = jnp.where(m, c_new, ct)


@functools.partial(jax.jit, static_argnames=("g",))
def _run(corr_index, rela_ht, rela_ct, nei_index,
         W_emb, b_emb, W_ih, W_hh, b_ih, b_hh, g=4):
    corr_t = jnp.transpose(corr_index, (0, 2, 1))
    nei3 = nei_index.reshape(P, 1, P)
    ht_t = jnp.transpose(rela_ht, (0, 2, 1))
    ct_t = jnp.transpose(rela_ct, (0, 2, 1))
    bemb = b_emb.reshape(E, 1)
    gate_scale = jnp.concatenate([
        jnp.full((2 * H, 1), 0.5, jnp.float32),
        jnp.ones((H, 1), jnp.float32),
        jnp.full((H, 1), 0.5, jnp.float32),
    ])
    wih_s = W_ih * gate_scale
    whh_s = W_hh * gate_scale
    bias = (b_ih + b_hh).reshape(4 * H, 1) * gate_scale

    grid = (P // g,)
    corr_spec = pl.BlockSpec((g, 2, P), lambda i: (i, 0, 0))
    nei_spec = pl.BlockSpec((g, 1, P), lambda i: (i, 0, 0))
    st_spec = pl.BlockSpec((g, H, P), lambda i: (i, 0, 0))
    full_spec = lambda r, c: pl.BlockSpec((r, c), lambda i: (0, 0))
    ho, co = pl.pallas_call(
        _lstm_slab_kernel,
        grid=grid,
        in_specs=[
            corr_spec,
            nei_spec,
            st_spec,
            st_spec,
            full_spec(E, 2),
            full_spec(E, 1),
            full_spec(4 * H, E),
            full_spec(4 * H, H),
            full_spec(4 * H, 1),
        ],
        out_specs=[st_spec, st_spec],
        out_shape=[
            jax.ShapeDtypeStruct((P, H, P), jnp.float32),
            jax.ShapeDtypeStruct((P, H, P), jnp.float32),
        ],
    )(corr_t, nei3, ht_t, ct_t, W_emb, bemb, wih_s, whh_s, bias)
    return jnp.transpose(ho, (0, 2, 1)), jnp.transpose(co, (0, 2, 1))


def kernel(corr_index, rela_ht, rela_ct, nei_index,
           W_emb, b_emb, W_ih, W_hh, b_ih, b_hh):
    return _run(corr_index, rela_ht, rela_ct, nei_index,
                W_emb, b_emb, W_ih, W_hh, b_ih, b_hh)


# speedup vs baseline: 1.2266x; 1.2266x over previous
import functools

import jax
import jax.numpy as jnp
from jax.experimental import pallas as pl

P = 512
H = 64
E = 32
N = P * P


def _lstm_slab_kernel(corr_ref, nei_ref, ht_ref, ct_ref,
                      wemb_ref, bemb_ref, wih_ref, whh_ref, bias_ref,
                      ho_ref, co_ref):
    g = ht_ref.shape[0]
    for j in range(g):
        corr = corr_ref[j]
        ht = ht_ref[j]
        ct = ct_ref[j]
        emb = jnp.maximum(
            jnp.dot(wemb_ref[...], corr, preferred_element_type=jnp.float32)
            + bemb_ref[...], 0.0)
        gates = (jnp.dot(wih_ref[...], emb, preferred_element_type=jnp.float32)
                 + jnp.dot(whh_ref[...], ht, preferred_element_type=jnp.float32)
                 + bias_ref[...])
        i = 0.5 * jnp.tanh(gates[0 * H:1 * H, :]) + 0.5
        f = 0.5 * jnp.tanh(gates[1 * H:2 * H, :]) + 0.5
        gg = jnp.tanh(gates[2 * H:3 * H, :])
        o = 0.5 * jnp.tanh(gates[3 * H:4 * H, :]) + 0.5
        c_new = f * ct + i * gg
        h_new = o * jnp.tanh(c_new)
        m = nei_ref[j] > 0
        ho_ref[j] = jnp.where(m, h_new, ht)
        co_ref[j] = jnp.where(m, c_new, ct)


@functools.partial(jax.jit, static_argnames=("g",))
def _run(corr_index, rela_ht, rela_ct, nei_index,
         W_emb, b_emb, W_ih, W_hh, b_ih, b_hh, g=8):
    corr_t = jnp.transpose(corr_index, (0, 2, 1))
    nei3 = nei_index.reshape(P, 1, P)
    ht_t = jnp.transpose(rela_ht, (0, 2, 1))
    ct_t = jnp.transpose(rela_ct, (0, 2, 1))
    bemb = b_emb.reshape(E, 1)
    gate_scale = jnp.concatenate([
        jnp.full((2 * H, 1), 0.5, jnp.float32),
        jnp.ones((H, 1), jnp.float32),
        jnp.full((H, 1), 0.5, jnp.float32),
    ])
    wih_s = W_ih * gate_scale
    whh_s = W_hh * gate_scale
    bias = (b_ih + b_hh).reshape(4 * H, 1) * gate_scale

    grid = (P // g,)
    corr_spec = pl.BlockSpec((g, 2, P), lambda i: (i, 0, 0))
    nei_spec = pl.BlockSpec((g, 1, P), lambda i: (i, 0, 0))
    st_spec = pl.BlockSpec((g, H, P), lambda i: (i, 0, 0))
    full_spec = lambda r, c: pl.BlockSpec((r, c), lambda i: (0, 0))
    ho, co = pl.pallas_call(
        _lstm_slab_kernel,
        grid=grid,
        in_specs=[
            corr_spec,
            nei_spec,
            st_spec,
            st_spec,
            full_spec(E, 2),
            full_spec(E, 1),
            full_spec(4 * H, E),
            full_spec(4 * H, H),
            full_spec(4 * H, 1),
        ],
        out_specs=[st_spec, st_spec],
        out_shape=[
            jax.ShapeDtypeStruct((P, H, P), jnp.float32),
            jax.ShapeDtypeStruct((P, H, P), jnp.float32),
        ],
    )(corr_t, nei3, ht_t, ct_t, W_emb, bemb, wih_s, whh_s, bias)
    return jnp.transpose(ho, (0, 2, 1)), jnp.transpose(co, (0, 2, 1))


def kernel(corr_index, rela_ht, rela_ct, nei_index,
           W_emb, b_emb, W_ih, W_hh, b_ih, b_hh):
    return _run(corr_index, rela_ht, rela_ct, nei_index,
                W_emb, b_emb, W_ih, W_hh, b_ih, b_hh)


# speedup vs baseline: 1.3124x; 1.0699x over previous
import functools

import jax
import jax.numpy as jnp
from jax.experimental import pallas as pl

P = 512
H = 64
E = 32
N = P * P


def _lstm_slab_kernel(corr_ref, nei_ref, ht_ref, ct_ref,
                      wemb_ref, bemb_ref, wih_ref, whh_ref, bias_ref,
                      ho_ref, co_ref):
    g = ht_ref.shape[0]
    for j in range(g):
        corr = corr_ref[j]
        ht = ht_ref[j]
        ct = ct_ref[j]
        emb = jnp.maximum(
            jnp.dot(wemb_ref[...], corr, preferred_element_type=jnp.float32)
            + bemb_ref[...], 0.0)
        gates = (jnp.dot(wih_ref[...], emb, preferred_element_type=jnp.float32)
                 + jnp.dot(whh_ref[...], ht, preferred_element_type=jnp.float32)
                 + bias_ref[...])
        i = 0.5 * jnp.tanh(gates[0 * H:1 * H, :]) + 0.5
        f = 0.5 * jnp.tanh(gates[1 * H:2 * H, :]) + 0.5
        gg = jnp.tanh(gates[2 * H:3 * H, :])
        o = 0.5 * jnp.tanh(gates[3 * H:4 * H, :]) + 0.5
        c_new = f * ct + i * gg
        h_new = o * jnp.tanh(c_new)
        m = nei_ref[j] > 0
        ho_ref[j] = jnp.where(m, h_new, ht)
        co_ref[j] = jnp.where(m, c_new, ct)


@functools.partial(jax.jit, static_argnames=("g",))
def _run(corr_index, rela_ht, rela_ct, nei_index,
         W_emb, b_emb, W_ih, W_hh, b_ih, b_hh, g=16):
    corr_t = jnp.transpose(corr_index, (0, 2, 1))
    nei3 = nei_index.reshape(P, 1, P)
    ht_t = jnp.transpose(rela_ht, (0, 2, 1))
    ct_t = jnp.transpose(rela_ct, (0, 2, 1))
    bemb = b_emb.reshape(E, 1)
    gate_scale = jnp.concatenate([
        jnp.full((2 * H, 1), 0.5, jnp.float32),
        jnp.ones((H, 1), jnp.float32),
        jnp.full((H, 1), 0.5, jnp.float32),
    ])
    wih_s = W_ih * gate_scale
    whh_s = W_hh * gate_scale
    bias = (b_ih + b_hh).reshape(4 * H, 1) * gate_scale

    grid = (P // g,)
    corr_spec = pl.BlockSpec((g, 2, P), lambda i: (i, 0, 0))
    nei_spec = pl.BlockSpec((g, 1, P), lambda i: (i, 0, 0))
    st_spec = pl.BlockSpec((g, H, P), lambda i: (i, 0, 0))
    full_spec = lambda r, c: pl.BlockSpec((r, c), lambda i: (0, 0))
    ho, co = pl.pallas_call(
        _lstm_slab_kernel,
        grid=grid,
        in_specs=[
            corr_spec,
            nei_spec,
            st_spec,
            st_spec,
            full_spec(E, 2),
            full_spec(E, 1),
            full_spec(4 * H, E),
            full_spec(4 * H, H),
            full_spec(4 * H, 1),
        ],
        out_specs=[st_spec, st_spec],
        out_shape=[
            jax.ShapeDtypeStruct((P, H, P), jnp.float32),
            jax.ShapeDtypeStruct((P, H, P), jnp.float32),
        ],
    )(corr_t, nei3, ht_t, ct_t, W_emb, bemb, wih_s, whh_s, bias)
    return jnp.transpose(ho, (0, 2, 1)), jnp.transpose(co, (0, 2, 1))


def kernel(corr_index, rela_ht, rela_ct, nei_index,
           W_emb, b_emb, W_ih, W_hh, b_ih, b_hh):
    return _run(corr_index, rela_ht, rela_ct, nei_index,
                W_emb, b_emb, W_ih, W_hh, b_ih, b_hh)


# speedup vs baseline: 1.3251x; 1.0097x over previous
import functools

import jax
import jax.numpy as jnp
from jax.experimental import pallas as pl

P = 512
H = 64
E = 32
N = P * P


def _lstm_slab_kernel(corr_ref, nei_ref, ht_ref, ct_ref,
                      wemb_ref, bemb_ref, wih_ref, whh_ref, bias_ref,
                      ho_ref, co_ref):
    g = ht_ref.shape[0]
    for j in range(g):
        corr = corr_ref[j]
        ht = ht_ref[j]
        ct = ct_ref[j]
        emb = jnp.maximum(
            jnp.dot(wemb_ref[...], corr, preferred_element_type=jnp.float32)
            + bemb_ref[...], 0.0)
        gates = (jnp.dot(wih_ref[...], emb, preferred_element_type=jnp.float32)
                 + jnp.dot(whh_ref[...], ht, preferred_element_type=jnp.float32)
                 + bias_ref[...])
        i = 0.5 * jnp.tanh(gates[0 * H:1 * H, :]) + 0.5
        f = 0.5 * jnp.tanh(gates[1 * H:2 * H, :]) + 0.5
        gg = jnp.tanh(gates[2 * H:3 * H, :])
        o = 0.5 * jnp.tanh(gates[3 * H:4 * H, :]) + 0.5
        c_new = f * ct + i * gg
        h_new = o * jnp.tanh(c_new)
        m = nei_ref[j] > 0
        ho_ref[j] = jnp.where(m, h_new, ht)
        co_ref[j] = jnp.where(m, c_new, ct)


@functools.partial(jax.jit, static_argnames=("g",))
def _run(corr_index, rela_ht, rela_ct, nei_index,
         W_emb, b_emb, W_ih, W_hh, b_ih, b_hh, g=32):
    corr_t = jnp.transpose(corr_index, (0, 2, 1))
    nei3 = nei_index.reshape(P, 1, P)
    ht_t = jnp.transpose(rela_ht, (0, 2, 1))
    ct_t = jnp.transpose(rela_ct, (0, 2, 1))
    bemb = b_emb.reshape(E, 1)
    gate_scale = jnp.concatenate([
        jnp.full((2 * H, 1), 0.5, jnp.float32),
        jnp.ones((H, 1), jnp.float32),
        jnp.full((H, 1), 0.5, jnp.float32),
    ])
    wih_s = W_ih * gate_scale
    whh_s = W_hh * gate_scale
    bias = (b_ih + b_hh).reshape(4 * H, 1) * gate_scale

    grid = (P // g,)
    corr_spec = pl.BlockSpec((g, 2, P), lambda i: (i, 0, 0))
    nei_spec = pl.BlockSpec((g, 1, P), lambda i: (i, 0, 0))
    st_spec = pl.BlockSpec((g, H, P), lambda i: (i, 0, 0))
    full_spec = lambda r, c: pl.BlockSpec((r, c), lambda i: (0, 0))
    ho, co = pl.pallas_call(
        _lstm_slab_kernel,
        grid=grid,
        in_specs=[
            corr_spec,
            nei_spec,
            st_spec,
            st_spec,
            full_spec(E, 2),
            full_spec(E, 1),
            full_spec(4 * H, E),
            full_spec(4 * H, H),
            full_spec(4 * H, 1),
        ],
        out_specs=[st_spec, st_spec],
        out_shape=[
            jax.ShapeDtypeStruct((P, H, P), jnp.float32),
            jax.ShapeDtypeStruct((P, H, P), jnp.float32),
        ],
    )(corr_t, nei3, ht_t, ct_t, W_emb, bemb, wih_s, whh_s, bias)
    return jnp.transpose(ho, (0, 2, 1)), jnp.transpose(co, (0, 2, 1))


def kernel(corr_index, rela_ht, rela_ct, nei_index,
           W_emb, b_emb, W_ih, W_hh, b_ih, b_hh):
    return _run(corr_index, rela_ht, rela_ct, nei_index,
                W_emb, b_emb, W_ih, W_hh, b_ih, b_hh)


# speedup vs baseline: 1.3462x; 1.0160x over previous
import functools

import jax
import jax.numpy as jnp
from jax.experimental import pallas as pl

P = 512
H = 64
E = 32
N = P * P


def _lstm_slab_kernel(corr_ref, nei_ref, ht_ref, ct_ref,
                      wemb_ref, bemb_ref, wih_ref, whh_ref, bias_ref,
                      ho_ref, co_ref):
    g = ht_ref.shape[0]
    for j in range(g):
        corr = corr_ref[j]
        ht = ht_ref[j]
        ct = ct_ref[j]
        emb = jnp.maximum(
            jnp.dot(wemb_ref[...], corr, preferred_element_type=jnp.float32)
            + bemb_ref[...], 0.0)
        gates = (jnp.dot(wih_ref[...], emb.astype(jnp.bfloat16),
                         preferred_element_type=jnp.float32)
                 + jnp.dot(whh_ref[...], ht.astype(jnp.bfloat16),
                           preferred_element_type=jnp.float32)
                 + bias_ref[...])
        i = 0.5 * jnp.tanh(gates[0 * H:1 * H, :]) + 0.5
        f = 0.5 * jnp.tanh(gates[1 * H:2 * H, :]) + 0.5
        gg = jnp.tanh(gates[2 * H:3 * H, :])
        o = 0.5 * jnp.tanh(gates[3 * H:4 * H, :]) + 0.5
        c_new = f * ct + i * gg
        h_new = o * jnp.tanh(c_new)
        m = nei_ref[j] > 0
        ho_ref[j] = jnp.where(m, h_new, ht)
        co_ref[j] = jnp.where(m, c_new, ct)


@functools.partial(jax.jit, static_argnames=("g",))
def _run(corr_index, rela_ht, rela_ct, nei_index,
         W_emb, b_emb, W_ih, W_hh, b_ih, b_hh, g=32):
    corr_t = jnp.transpose(corr_index, (0, 2, 1))
    nei3 = nei_index.reshape(P, 1, P)
    ht_t = jnp.transpose(rela_ht, (0, 2, 1))
    ct_t = jnp.transpose(rela_ct, (0, 2, 1))
    bemb = b_emb.reshape(E, 1)
    gate_scale = jnp.concatenate([
        jnp.full((2 * H, 1), 0.5, jnp.float32),
        jnp.ones((H, 1), jnp.float32),
        jnp.full((H, 1), 0.5, jnp.float32),
    ])
    wih_s = (W_ih * gate_scale).astype(jnp.bfloat16)
    whh_s = (W_hh * gate_scale).astype(jnp.bfloat16)
    bias = (b_ih + b_hh).reshape(4 * H, 1) * gate_scale

    grid = (P // g,)
    corr_spec = pl.BlockSpec((g, 2, P), lambda i: (i, 0, 0))
    nei_spec = pl.BlockSpec((g, 1, P), lambda i: (i, 0, 0))
    st_spec = pl.BlockSpec((g, H, P), lambda i: (i, 0, 0))
    full_spec = lambda r, c: pl.BlockSpec((r, c), lambda i: (0, 0))
    ho, co = pl.pallas_call(
        _lstm_slab_kernel,
        grid=grid,
        in_specs=[
            corr_spec,
            nei_spec,
            st_spec,
            st_spec,
            full_spec(E, 2),
            full_spec(E, 1),
            full_spec(4 * H, E),
            full_spec(4 * H, H),
            full_spec(4 * H, 1),
        ],
        out_specs=[st_spec, st_spec],
        out_shape=[
            jax.ShapeDtypeStruct((P, H, P), jnp.float32),
            jax.ShapeDtypeStruct((P, H, P), jnp.float32),
        ],
    )(corr_t, nei3, ht_t, ct_t, W_emb, bemb, wih_s, whh_s, bias)
    return jnp.transpose(ho, (0, 2, 1)), jnp.transpose(co, (0, 2, 1))


def kernel(corr_index, rela_ht, rela_ct, nei_index,
           W_emb, b_emb, W_ih, W_hh, b_ih, b_hh):
    return _run(corr_index, rela_ht, rela_ct, nei_index,
                W_emb, b_emb, W_ih, W_hh, b_ih, b_hh)
